# Initial kernel scaffold; baseline (speedup 1.0000x reference)
import jax
import jax.numpy as jnp
from jax.experimental import pallas as pl


def kernel(x, edge_label_index):
    raise NotImplementedError("write your pallas kernel here")


# speedup vs baseline: 5.9105x; 5.9105x over previous
import functools

import jax
import jax.numpy as jnp
from jax import lax
from jax.experimental import pallas as pl
from jax.experimental.pallas import tpu as pltpu
from jax.experimental.pallas import tpu_sc as plsc

NC = 2
NS = 16
NW = NC * NS
L = 16

CH = 80


def _make_kernel(E, D):
    H = D // 2
    EPW = E // NW
    NCH = EPW // CH
    mesh = plsc.VectorSubcoreMesh(core_axis_name="c", subcore_axis_name="s",
                                  num_cores=NC)

    @functools.partial(
        pl.kernel,
        mesh=mesh,
        compiler_params=pltpu.CompilerParams(needs_layout_passes=False),
        out_type=jax.ShapeDtypeStruct((E,), jnp.float32),
        scratch_types=[
            pltpu.VMEM((EPW,), jnp.int32),
            pltpu.VMEM((EPW,), jnp.int32),
            pltpu.VMEM((EPW,), jnp.float32),
            pltpu.VMEM((CH, D), jnp.float32),
            pltpu.VMEM((CH, D), jnp.float32),
            pltpu.SemaphoreType.DMA,
            pltpu.SemaphoreType.DMA,
        ],
    )
    def decode(x_hbm, src_hbm, dst_hbm, out_hbm, idx_s, idx_d, out_v,
               rows_s, rows_d, sem_s, sem_d):
        wid = lax.axis_index("s") * NC + lax.axis_index("c")
        base = wid * EPW

        pltpu.sync_copy(src_hbm.at[pl.ds(base, EPW)], idx_s)
        pltpu.sync_copy(dst_hbm.at[pl.ds(base, EPW)], idx_d)

        def chunk(g, carry):
            goff = g * CH
            cp_s = pltpu.make_async_copy(
                x_hbm.at[idx_s.at[pl.ds(goff, CH)]], rows_s, sem_s)
            cp_d = pltpu.make_async_copy(
                x_hbm.at[idx_d.at[pl.ds(goff, CH)]], rows_d, sem_d)
            cp_s.start()
            cp_d.start()
            cp_s.wait()
            cp_d.wait()
            lane = jnp.arange(L, dtype=jnp.int32)
            for jj in range(CH // L):
                vec = jnp.zeros((L,), jnp.float32)
                for t in range(L):
                    j = jj * L + t
                    acc = rows_s[j, pl.ds(0, L)] * rows_d[j, pl.ds(H, L)]
                    for k in range(1, H // L):
                        acc += (rows_s[j, pl.ds(k * L, L)]
                                * rows_d[j, pl.ds(H + k * L, L)])
                    vec = jnp.where(lane == t, jnp.sum(acc), vec)
                out_v[pl.ds(goff + jj * L, L)] = vec
            return carry

        lax.fori_loop(0, NCH, chunk, 0)

        def sigmoid(i, carry):
            v = out_v[pl.ds(i * L, L)]
            out_v[pl.ds(i * L, L)] = 1.0 / (1.0 + jnp.exp(-v))
            return carry

        lax.fori_loop(0, EPW // L, sigmoid, 0)

        pltpu.sync_copy(out_v, out_hbm.at[pl.ds(base, EPW)])

    return decode


def kernel(x, edge_label_index):
    N, D = x.shape
    E = edge_label_index.shape[1]
    out = _make_kernel(E, D)(x, edge_label_index[0], edge_label_index[1])
    return out.reshape(E, 1)


# speedup vs baseline: 6.9182x; 1.1705x over previous
import functools

import jax
import jax.numpy as jnp
from jax import lax
from jax.experimental import pallas as pl
from jax.experimental.pallas import tpu as pltpu
from jax.experimental.pallas import tpu_sc as plsc

NC = 2
NS = 16
NW = NC * NS
L = 16

CH = 80


def _make_kernel(E, D):
    H = D // 2
    EPW = E // NW
    NCH = EPW // CH
    assert E % NW == 0 and EPW % CH == 0 and NCH % 2 == 1
    mesh = plsc.VectorSubcoreMesh(core_axis_name="c", subcore_axis_name="s",
                                  num_cores=NC)

    @functools.partial(
        pl.kernel,
        mesh=mesh,
        compiler_params=pltpu.CompilerParams(needs_layout_passes=False),
        out_type=jax.ShapeDtypeStruct((E,), jnp.float32),
        scratch_types=[
            pltpu.VMEM((EPW,), jnp.int32),
            pltpu.VMEM((EPW,), jnp.int32),
            pltpu.VMEM((EPW,), jnp.float32),
            pltpu.VMEM((CH, D), jnp.float32),
            pltpu.VMEM((CH, D), jnp.float32),
            pltpu.VMEM((CH, D), jnp.float32),
            pltpu.VMEM((CH, D), jnp.float32),
            pltpu.SemaphoreType.DMA,
            pltpu.SemaphoreType.DMA,
            pltpu.SemaphoreType.DMA,
            pltpu.SemaphoreType.DMA,
        ],
    )
    def decode(x_hbm, src_hbm, dst_hbm, out_hbm, idx_s, idx_d, out_v,
               rows_sa, rows_da, rows_sb, rows_db,
               sem_sa, sem_da, sem_sb, sem_db):
        wid = lax.axis_index("s") * NC + lax.axis_index("c")
        base = wid * EPW

        pltpu.sync_copy(src_hbm.at[pl.ds(base, EPW)], idx_s)
        pltpu.sync_copy(dst_hbm.at[pl.ds(base, EPW)], idx_d)

        def start(goff, rows_s, rows_d, sem_s, sem_d):
            cp_s = pltpu.make_async_copy(
                x_hbm.at[idx_s.at[pl.ds(goff, CH)]], rows_s, sem_s)
            cp_d = pltpu.make_async_copy(
                x_hbm.at[idx_d.at[pl.ds(goff, CH)]], rows_d, sem_d)
            cp_s.start()
            cp_d.start()
            return cp_s, cp_d

        def compute(goff, rows_s, rows_d):
            lane = jnp.arange(L, dtype=jnp.int32)
            for jj in range(CH // L):
                vec = jnp.zeros((L,), jnp.float32)
                for t in range(L):
                    j = jj * L + t
                    acc = rows_s[j, pl.ds(0, L)] * rows_d[j, pl.ds(H, L)]
                    for k in range(1, H // L):
                        acc += (rows_s[j, pl.ds(k * L, L)]
                                * rows_d[j, pl.ds(H + k * L, L)])
                    vec = jnp.where(lane == t, jnp.sum(acc), vec)
                out_v[pl.ds(goff + jj * L, L)] = vec

        start(0, rows_sa, rows_da, sem_sa, sem_da)

        def pair(i, carry):
            goff_a = (2 * i) * CH
            start(goff_a + CH, rows_sb, rows_db, sem_sb, sem_db)
            pltpu.make_async_copy(
                x_hbm.at[idx_s.at[pl.ds(goff_a, CH)]], rows_sa, sem_sa).wait()
            pltpu.make_async_copy(
                x_hbm.at[idx_d.at[pl.ds(goff_a, CH)]], rows_da, sem_da).wait()
            compute(goff_a, rows_sa, rows_da)
            start(goff_a + 2 * CH, rows_sa, rows_da, sem_sa, sem_da)
            pltpu.make_async_copy(
                x_hbm.at[idx_s.at[pl.ds(goff_a, CH)]], rows_sb, sem_sb).wait()
            pltpu.make_async_copy(
                x_hbm.at[idx_d.at[pl.ds(goff_a, CH)]], rows_db, sem_db).wait()
            compute(goff_a + CH, rows_sb, rows_db)
            return carry

        lax.fori_loop(0, (NCH - 1) // 2, pair, 0)
        pltpu.make_async_copy(
            x_hbm.at[idx_s.at[pl.ds(0, CH)]], rows_sa, sem_sa).wait()
        pltpu.make_async_copy(
            x_hbm.at[idx_d.at[pl.ds(0, CH)]], rows_da, sem_da).wait()
        compute((NCH - 1) * CH, rows_sa, rows_da)

        def sigmoid(i, carry):
            v = out_v[pl.ds(i * L, L)]
            out_v[pl.ds(i * L, L)] = 1.0 / (1.0 + jnp.exp(-v))
            return carry

        lax.fori_loop(0, EPW // L, sigmoid, 0)

        pltpu.sync_copy(out_v, out_hbm.at[pl.ds(base, EPW)])

    return decode


def kernel(x, edge_label_index):
    N, D = x.shape
    E = edge_label_index.shape[1]
    out = _make_kernel(E, D)(x, edge_label_index[0], edge_label_index[1])
    return out.reshape(E, 1)
